# baseline (speedup 1.0000x reference)
import functools

import jax
import jax.numpy as jnp
from jax import lax
from jax.experimental import pallas as pl
from jax.experimental.pallas import tpu as pltpu
from jax.experimental.pallas import tpu_sc as plsc

R = 128
N = 32768
K = (N - 1) // 2
L = 16
NW = 32
RPW = R // NW
U = 4
PAD = L * U
HI0 = 0x3F7FFFFF
MAXIT = 64

_mesh = plsc.VectorSubcoreMesh(core_axis_name="c", subcore_axis_name="s")


def _xsum(v):
    return plsc.cumsum(v)[L - 1]


@functools.partial(
    pl.kernel,
    mesh=_mesh,
    out_type=jax.ShapeDtypeStruct((NW, L), jnp.float32),
    compiler_params=pltpu.CompilerParams(needs_layout_passes=False),
    scratch_types=[
        pltpu.VMEM((N + PAD,), jnp.float32),
        pltpu.VMEM((N + PAD,), jnp.float32),
        pltpu.VMEM((L,), jnp.float32),
    ],
)
def _soft_majority_sc(x_hbm, out_hbm, buf_a, buf_b, res_v):
    wid = lax.axis_index("s") * 2 + lax.axis_index("c")
    lane = lax.iota(jnp.int32, L)
    zero_i = jnp.zeros((L,), jnp.int32)

    def split(src, dst, start_s, size_s, probe_f, with_sum):
        size_v = jnp.broadcast_to(size_s, (L,))
        size_m1 = size_v - 1
        nit = (size_s + (L * U - 1)) // (L * U)

        def body(i, carry):
            if with_sum:
                offL, offH, acc = carry
            else:
                offL, offH = carry
            b = i * (L * U)
            for u in range(U):
                v = src[pl.ds(start_s + b + u * L, L)]
                pos = (b + u * L) + lane
                valid = pos < size_v
                le = v <= probe_f
                m = jnp.logical_and(le, valid)
                mh = jnp.logical_and(jnp.logical_not(le), valid)
                mi = jnp.where(m, 1, 0).astype(jnp.int32)
                hv = jnp.where(mh, 1, 0).astype(jnp.int32)
                exclL = plsc.cumsum(mi) - mi
                exclH = plsc.cumsum(hv) - hv
                idxL = offL + exclL
                idxH = jnp.maximum(size_m1 - offH - exclH, 0)
                plsc.store_scatter(dst, [idxL], v, mask=m)
                plsc.store_scatter(dst, [idxH], v, mask=mh)
                offL = offL + plsc.all_reduce_population_count(m)
                offH = offH + plsc.all_reduce_population_count(mh)
                if with_sum:
                    acc = acc + jnp.where(valid, v, 0.0)
            return (offL, offH, acc) if with_sum else (offL, offH)

        init = (zero_i, zero_i, jnp.zeros((L,), jnp.float32)) if with_sum \
            else (zero_i, zero_i)
        out = lax.fori_loop(0, nit, body, init)
        if with_sum:
            return out[0][0], out[1][0], out[2]
        return out[0][0], out[1][0]

    def decide(go, k_s, cL, cH, lo_v, hi_v, probe_pat):
        go_v = jnp.broadcast_to(go, (L,))
        k2 = jnp.where(go, k_s, k_s - cL)
        start2 = jnp.where(go, 0, cL)
        size2 = jnp.where(go, cL, cH)
        hi2 = jnp.where(go_v, probe_pat, hi_v)
        lo2 = jnp.where(go_v, lo_v, probe_pat + 1)
        return k2, start2, size2, lo2, hi2

    def finish(src, start_s, size_s, k_s):
        v = src[pl.ds(start_s, L)]
        valid = lane < jnp.broadcast_to(size_s, (L,))
        skeys, _, _ = plsc.sort_key_val(v, v, mask=valid)
        sel = jnp.where(lane == jnp.broadcast_to(k_s, (L,)), skeys, 0.0)
        return _xsum(sel)

    def process_row(jj, res):
        row = wid * RPW + jj
        pltpu.sync_copy(x_hbm.at[row], buf_a.at[pl.ds(0, N)])

        probe0 = jnp.full((L,), 0.5, jnp.float32)
        cL, cH, acc = split(buf_a, buf_b, 0, N, probe0, True)
        mean = _xsum(acc) * (1.0 / N)
        k1, start1, size1, lo1, hi1 = decide(
            K < cL, jnp.int32(K), cL, cH,
            zero_i, jnp.full((L,), HI0, jnp.int32),
            jnp.full((L,), 0x3F000000, jnp.int32))

        def w_cond(st):
            start_s, size_s, k_s, parity, it, lo_v, hi_v = st
            return (size_s > L) & (lo_v[0] < hi_v[0]) & (it < MAXIT)

        def w_body(st):
            start_s, size_s, k_s, parity, it, lo_v, hi_v = st
            lo_f = lax.bitcast_convert_type(lo_v, jnp.float32)
            hi_f = lax.bitcast_convert_type(hi_v, jnp.float32)
            vmid = 0.5 * (lo_f + hi_f)
            pmid = lax.bitcast_convert_type(vmid, jnp.int32)
            probe_pat = jnp.minimum(jnp.maximum(pmid, lo_v), hi_v - 1)
            probe_f = lax.bitcast_convert_type(probe_pat, jnp.float32)
            cL2, cH2 = lax.cond(
                parity == 0,
                lambda: split(buf_b, buf_a, start_s, size_s, probe_f, False),
                lambda: split(buf_a, buf_b, start_s, size_s, probe_f, False))
            k2, start2, size2, lo2, hi2 = decide(
                k_s < cL2, k_s, cL2, cH2, lo_v, hi_v, probe_pat)
            return (start2, size2, k2, 1 - parity, it + 1, lo2, hi2)

        start_s, size_s, k_s, parity, _, lo_v, _ = lax.while_loop(
            w_cond, w_body,
            (start1, size1, k1, jnp.int32(0), jnp.int32(1), lo1, hi1))

        m_sorted = lax.cond(
            parity == 0,
            lambda: finish(buf_b, start_s, size_s, k_s),
            lambda: finish(buf_a, start_s, size_s, k_s))
        m_bit = jnp.where(size_s <= L, m_sorted,
                          lax.bitcast_convert_type(lo_v, jnp.float32)[0])

        margin = jnp.abs(m_bit - 0.5)
        md = mean * margin
        rep = jnp.where(m_bit > 0.5, 0.5 + md, m_bit + md)
        return jnp.where(lane == jnp.broadcast_to(jj, (L,)),
                         jnp.broadcast_to(rep, (L,)), res)

    res = lax.fori_loop(0, RPW, process_row, jnp.zeros((L,), jnp.float32))
    res_v[...] = res
    pltpu.sync_copy(res_v, out_hbm.at[wid])


def kernel(x):
    padded = _soft_majority_sc(x)
    return padded[:, :RPW].reshape(R)


# speedup vs baseline: 1.0708x; 1.0708x over previous
import functools

import jax
import jax.numpy as jnp
from jax import lax
from jax.experimental import pallas as pl
from jax.experimental.pallas import tpu as pltpu
from jax.experimental.pallas import tpu_sc as plsc

R = 128
N = 32768
K = (N - 1) // 2
L = 16
NW = 32
RPW = R // NW
U = 4
PAD = L * U
HI0 = 0x3F7FFFFF
MAXIT = 64

_mesh = plsc.VectorSubcoreMesh(core_axis_name="c", subcore_axis_name="s")


def _xsum(v):
    return plsc.cumsum(v)[L - 1]


@functools.partial(
    pl.kernel,
    mesh=_mesh,
    out_type=jax.ShapeDtypeStruct((NW, L), jnp.float32),
    compiler_params=pltpu.CompilerParams(needs_layout_passes=False),
    scratch_types=[
        pltpu.VMEM((N + PAD,), jnp.float32),
        pltpu.VMEM((N + PAD,), jnp.float32),
        pltpu.VMEM((N + PAD,), jnp.float32),
        pltpu.VMEM((L,), jnp.float32),
    ],
)
def _soft_majority_sc(x_hbm, out_hbm, buf0, buf1, buf2, res_v):
    wid = lax.axis_index("s") * 2 + lax.axis_index("c")
    lane = lax.iota(jnp.int32, L)
    zero_i = jnp.zeros((L,), jnp.int32)
    bufs = (buf0, buf1, buf2)

    def split(src, d_lo, d_hi, size_s, probe_f, with_sum):
        nfull = size_s // (L * U)

        def full_body(i, carry):
            if with_sum:
                offL, offH, acc = carry
            else:
                offL, offH = carry
            b = i * (L * U)
            for u in range(U):
                v = src[pl.ds(b + u * L, L)]
                le = v <= probe_f
                gt = jnp.logical_not(le)
                plsc.store_compressed(d_lo.at[pl.ds(offL, L)], v, mask=le)
                plsc.store_compressed(d_hi.at[pl.ds(offH, L)], v, mask=gt)
                pc = plsc.all_reduce_population_count(le)[0]
                offL = offL + pc
                offH = offH + (L - pc)
                if with_sum:
                    acc = acc + v
            return (offL, offH, acc) if with_sum else (offL, offH)

        init = (jnp.int32(0), jnp.int32(0))
        if with_sum:
            init = init + (jnp.zeros((L,), jnp.float32),)
        out = lax.fori_loop(0, nfull, full_body, init)
        offL, offH = out[0], out[1]
        acc = out[2] if with_sum else None

        size_v = jnp.broadcast_to(size_s, (L,))
        b = nfull * (L * U)
        for u in range(U):
            base = b + u * L
            v = src[pl.ds(base, L)]
            valid = (base + lane) < size_v
            le0 = v <= probe_f
            le = jnp.logical_and(le0, valid)
            gt = jnp.logical_and(jnp.logical_not(le0), valid)
            plsc.store_compressed(d_lo.at[pl.ds(offL, L)], v, mask=le)
            plsc.store_compressed(d_hi.at[pl.ds(offH, L)], v, mask=gt)
            offL = offL + plsc.all_reduce_population_count(le)[0]
            offH = offH + plsc.all_reduce_population_count(gt)[0]
            if with_sum:
                acc = acc + jnp.where(valid, v, 0.0)
        if with_sum:
            return offL, offH, acc
        return offL, offH

    def split_from(src_id, size_s, probe_f):
        return lax.cond(
            src_id == 0,
            lambda: split(buf0, buf1, buf2, size_s, probe_f, False),
            lambda: lax.cond(
                src_id == 1,
                lambda: split(buf1, buf0, buf2, size_s, probe_f, False),
                lambda: split(buf2, buf0, buf1, size_s, probe_f, False)))

    def finish(src_id, size_s, k_s):
        def fin(src):
            v = src[pl.ds(0, L)]
            valid = lane < jnp.broadcast_to(size_s, (L,))
            skeys, _, _ = plsc.sort_key_val(v, v, mask=valid)
            sel = jnp.where(lane == jnp.broadcast_to(k_s, (L,)), skeys, 0.0)
            return _xsum(sel)
        return lax.cond(
            src_id == 0, lambda: fin(buf0),
            lambda: lax.cond(src_id == 1, lambda: fin(buf1),
                             lambda: fin(buf2)))

    def decide(go, k_s, cL, cH, lo_v, hi_v, probe_pat):
        go_v = jnp.broadcast_to(go, (L,))
        k2 = jnp.where(go, k_s, k_s - cL)
        size2 = jnp.where(go, cL, cH)
        hi2 = jnp.where(go_v, probe_pat, hi_v)
        lo2 = jnp.where(go_v, lo_v, probe_pat + 1)
        return k2, size2, lo2, hi2

    def process_row(jj, res):
        row = wid * RPW + jj
        pltpu.sync_copy(x_hbm.at[row], buf0.at[pl.ds(0, N)])

        probe0 = jnp.full((L,), 0.5, jnp.float32)
        cL, cH, acc = split(buf0, buf1, buf2, N, probe0, True)
        mean = _xsum(acc) * (1.0 / N)
        go = K < cL
        k1, size1, lo1, hi1 = decide(
            go, jnp.int32(K), cL, cH,
            zero_i, jnp.full((L,), HI0, jnp.int32),
            jnp.full((L,), 0x3F000000, jnp.int32))
        src1 = jnp.where(go, 1, 2)

        def w_cond(st):
            src_id, size_s, k_s, it, lo_v, hi_v = st
            return (size_s > L) & (lo_v[0] < hi_v[0]) & (it < MAXIT)

        def w_body(st):
            src_id, size_s, k_s, it, lo_v, hi_v = st
            lo_f = lax.bitcast_convert_type(lo_v, jnp.float32)
            hi_f = lax.bitcast_convert_type(hi_v, jnp.float32)
            vmid = 0.5 * (lo_f + hi_f)
            pmid = lax.bitcast_convert_type(vmid, jnp.int32)
            probe_pat = jnp.minimum(jnp.maximum(pmid, lo_v), hi_v - 1)
            probe_f = lax.bitcast_convert_type(probe_pat, jnp.float32)
            cL2, cH2 = split_from(src_id, size_s, probe_f)
            go2 = k_s < cL2
            k2, size2, lo2, hi2 = decide(go2, k_s, cL2, cH2,
                                         lo_v, hi_v, probe_pat)
            lodest = jnp.where(src_id == 0, 1, 0)
            hidest = jnp.where(src_id == 2, 1, 2)
            src2 = jnp.where(go2, lodest, hidest)
            return (src2, size2, k2, it + 1, lo2, hi2)

        src_id, size_s, k_s, _, lo_v, _ = lax.while_loop(
            w_cond, w_body, (src1, size1, k1, jnp.int32(1), lo1, hi1))

        m_sorted = finish(src_id, size_s, k_s)
        m_bit = jnp.where(size_s <= L, m_sorted,
                          lax.bitcast_convert_type(lo_v, jnp.float32)[0])

        margin = jnp.abs(m_bit - 0.5)
        md = mean * margin
        rep = jnp.where(m_bit > 0.5, 0.5 + md, m_bit + md)
        return jnp.where(lane == jnp.broadcast_to(jj, (L,)),
                         jnp.broadcast_to(rep, (L,)), res)

    res = lax.fori_loop(0, RPW, process_row, jnp.zeros((L,), jnp.float32))
    res_v[...] = res
    pltpu.sync_copy(res_v, out_hbm.at[wid])


def kernel(x):
    padded = _soft_majority_sc(x)
    return padded[:, :RPW].reshape(R)


# speedup vs baseline: 1.5021x; 1.4028x over previous
import functools

import jax
import jax.numpy as jnp
from jax import lax
from jax.experimental import pallas as pl
from jax.experimental.pallas import tpu as pltpu
from jax.experimental.pallas import tpu_sc as plsc

R = 128
N = 32768
K = (N - 1) // 2
L = 16
NW = 32
RPW = R // NW
U = 4
SEG = N // L
REGS = SEG + 1
BLEN = (L - 1) * REGS + SEG + L
HI0 = 0x3F7FFFFF
MAXIT = 64

_mesh = plsc.VectorSubcoreMesh(core_axis_name="c", subcore_axis_name="s")


def _xsum(v):
    return plsc.cumsum(v)[L - 1]


@functools.partial(
    pl.kernel,
    mesh=_mesh,
    out_type=jax.ShapeDtypeStruct((NW, L), jnp.float32),
    compiler_params=pltpu.CompilerParams(needs_layout_passes=False),
    scratch_types=[
        pltpu.VMEM((BLEN,), jnp.float32),
        pltpu.VMEM((BLEN,), jnp.float32),
        pltpu.VMEM((BLEN,), jnp.float32),
        pltpu.VMEM((2 * L,), jnp.float32),
        pltpu.VMEM((L,), jnp.float32),
    ],
)
def _soft_majority_sc(x_hbm, out_hbm, buf0, buf1, buf2, tiny_v, res_v):
    wid = lax.axis_index("s") * 2 + lax.axis_index("c")
    lane = lax.iota(jnp.int32, L)
    zero_i = jnp.zeros((L,), jnp.int32)
    one_i = jnp.ones((L,), jnp.int32)
    base_v = lane * REGS

    def split0(src, d_lo, d_hi, probe_f):
        def body(i, carry):
            offL, offH, acc = carry
            b = i * (L * U)
            for u in range(U):
                v = src[pl.ds(b + u * L, L)]
                le = v <= probe_f
                mi = jnp.where(le, one_i, zero_i)
                plsc.store_scatter(d_lo, [offL], v, mask=le)
                plsc.store_scatter(d_hi, [offH], v,
                                   mask=jnp.logical_not(le))
                offL = offL + mi
                offH = offH + (one_i - mi)
                acc = acc + v
            return (offL, offH, acc)

        offL, offH, acc = lax.fori_loop(
            0, SEG // U, body,
            (base_v, base_v, jnp.zeros((L,), jnp.float32)))
        return offL - base_v, offH - base_v, acc

    def splitn(src, d_lo, d_hi, s_vec, probe_f):
        trip = plsc.cummax(s_vec)[L - 1]

        def body(i, carry):
            offL, offH = carry
            for u in range(U):
                iu = i * U + u
                idx = base_v + iu
                v = plsc.load_gather(src, [idx])
                valid = s_vec > iu
                le0 = v <= probe_f
                m = jnp.logical_and(le0, valid)
                mh = jnp.logical_and(jnp.logical_not(le0), valid)
                plsc.store_scatter(d_lo, [offL], v, mask=m)
                plsc.store_scatter(d_hi, [offH], v, mask=mh)
                offL = offL + jnp.where(m, one_i, zero_i)
                offH = offH + jnp.where(mh, one_i, zero_i)
            return (offL, offH)

        nit = (trip + (U - 1)) // U
        offL, offH = lax.fori_loop(0, nit, body, (base_v, base_v))
        return offL - base_v, offH - base_v

    def split_from(src_id, s_vec, probe_f):
        return lax.cond(
            src_id == 0,
            lambda: splitn(buf0, buf1, buf2, s_vec, probe_f),
            lambda: lax.cond(
                src_id == 1,
                lambda: splitn(buf1, buf0, buf2, s_vec, probe_f),
                lambda: splitn(buf2, buf0, buf1, s_vec, probe_f)))

    def collect(src_id, s_vec, size_s, k_s):
        def coll(src):
            off = jnp.int32(0)
            for i in range(L):
                v = plsc.load_gather(src, [base_v + i])
                valid = s_vec > i
                plsc.store_compressed(tiny_v.at[pl.ds(off, L)], v,
                                      mask=valid)
                off = off + plsc.all_reduce_population_count(valid)[0]
            w = tiny_v[pl.ds(0, L)]
            valid2 = lane < jnp.broadcast_to(size_s, (L,))
            skeys, _, _ = plsc.sort_key_val(w, w, mask=valid2)
            sel = jnp.where(lane == jnp.broadcast_to(k_s, (L,)), skeys, 0.0)
            return _xsum(sel)
        return lax.cond(
            src_id == 0, lambda: coll(buf0),
            lambda: lax.cond(src_id == 1, lambda: coll(buf1),
                             lambda: coll(buf2)))

    def decide(go, k_s, cL, size_s, sL, sH, lo_v, hi_v, probe_pat):
        go_v = jnp.broadcast_to(go, (L,))
        k2 = jnp.where(go, k_s, k_s - cL)
        size2 = jnp.where(go, cL, size_s - cL)
        s2 = jnp.where(go_v, sL, sH)
        hi2 = jnp.where(go_v, probe_pat, hi_v)
        lo2 = jnp.where(go_v, lo_v, probe_pat + 1)
        return k2, size2, s2, lo2, hi2

    def process_row(jj, res):
        row = wid * RPW + jj
        pltpu.sync_copy(x_hbm.at[row], buf0.at[pl.ds(0, N)])

        probe0 = jnp.full((L,), 0.5, jnp.float32)
        sL, sH, acc = split0(buf0, buf1, buf2, probe0)
        mean = _xsum(acc) * (1.0 / N)
        cL = _xsum(sL)
        go = K < cL
        k1, size1, s1, lo1, hi1 = decide(
            go, jnp.int32(K), cL, jnp.int32(N), sL, sH,
            zero_i, jnp.full((L,), HI0, jnp.int32),
            jnp.full((L,), 0x3F000000, jnp.int32))
        src1 = jnp.where(go, 1, 2)

        def w_cond(st):
            src_id, s_vec, size_s, k_s, it, lo_v, hi_v = st
            return (size_s > L) & (lo_v[0] < hi_v[0]) & (it < MAXIT)

        def w_body(st):
            src_id, s_vec, size_s, k_s, it, lo_v, hi_v = st
            lo_f = lax.bitcast_convert_type(lo_v, jnp.float32)
            hi_f = lax.bitcast_convert_type(hi_v, jnp.float32)
            vmid = 0.5 * (lo_f + hi_f)
            pmid = lax.bitcast_convert_type(vmid, jnp.int32)
            probe_pat = jnp.minimum(jnp.maximum(pmid, lo_v), hi_v - 1)
            probe_f = lax.bitcast_convert_type(probe_pat, jnp.float32)
            sL2, sH2 = split_from(src_id, s_vec, probe_f)
            cL2 = _xsum(sL2)
            go2 = k_s < cL2
            k2, size2, s2, lo2, hi2 = decide(
                go2, k_s, cL2, size_s, sL2, sH2, lo_v, hi_v, probe_pat)
            lodest = jnp.where(src_id == 0, 1, 0)
            hidest = jnp.where(src_id == 2, 1, 2)
            src2 = jnp.where(go2, lodest, hidest)
            return (src2, s2, size2, k2, it + 1, lo2, hi2)

        src_id, s_vec, size_s, k_s, _, lo_v, _ = lax.while_loop(
            w_cond, w_body,
            (src1, s1, size1, k1, jnp.int32(1), lo1, hi1))

        m_sorted = lax.cond(
            size_s <= L,
            lambda: collect(src_id, s_vec, size_s, k_s),
            lambda: jnp.float32(0.0))
        m_bit = jnp.where(size_s <= L, m_sorted,
                          lax.bitcast_convert_type(lo_v, jnp.float32)[0])

        margin = jnp.abs(m_bit - 0.5)
        md = mean * margin
        rep = jnp.where(m_bit > 0.5, 0.5 + md, m_bit + md)
        return jnp.where(lane == jnp.broadcast_to(jj, (L,)),
                         jnp.broadcast_to(rep, (L,)), res)

    res = lax.fori_loop(0, RPW, process_row, jnp.zeros((L,), jnp.float32))
    res_v[...] = res
    pltpu.sync_copy(res_v, out_hbm.at[wid])


def kernel(x):
    padded = _soft_majority_sc(x)
    return padded[:, :RPW].reshape(R)


# speedup vs baseline: 2.0211x; 1.3455x over previous
import functools

import jax
import jax.numpy as jnp
from jax import lax
from jax.experimental import pallas as pl
from jax.experimental.pallas import tpu as pltpu
from jax.experimental.pallas import tpu_sc as plsc

R = 128
N = 32768
K = (N - 1) // 2
L = 16
NW = 32
RPW = R // NW
UC = 8
U = 4
NCOUNT = 5
SEG = N // L
REGS = SEG + 1
BLEN = (L - 1) * REGS + SEG + L
HI0 = 0x3F7FFFFF
MAXIT = 64

_mesh = plsc.VectorSubcoreMesh(core_axis_name="c", subcore_axis_name="s")


def _xsum(v):
    return plsc.cumsum(v)[L - 1]


@functools.partial(
    pl.kernel,
    mesh=_mesh,
    out_type=jax.ShapeDtypeStruct((NW, L), jnp.float32),
    compiler_params=pltpu.CompilerParams(needs_layout_passes=False),
    scratch_types=[
        pltpu.VMEM((BLEN,), jnp.float32),
        pltpu.VMEM((BLEN,), jnp.float32),
        pltpu.VMEM((BLEN,), jnp.float32),
        pltpu.VMEM((2 * L,), jnp.float32),
        pltpu.VMEM((L,), jnp.float32),
    ],
)
def _soft_majority_sc(x_hbm, out_hbm, buf0, buf1, buf2, tiny_v, res_v):
    wid = lax.axis_index("s") * 2 + lax.axis_index("c")
    lane = lax.iota(jnp.int32, L)
    zero_i = jnp.zeros((L,), jnp.int32)
    one_i = jnp.ones((L,), jnp.int32)
    base_v = lane * REGS

    def mk_probe(lo_v, hi_v):
        lo_f = lax.bitcast_convert_type(lo_v, jnp.float32)
        hi_f = lax.bitcast_convert_type(hi_v, jnp.float32)
        vmid = 0.5 * (lo_f + hi_f)
        pmid = lax.bitcast_convert_type(vmid, jnp.int32)
        probe_pat = jnp.minimum(jnp.maximum(pmid, lo_v), hi_v - 1)
        return probe_pat, lax.bitcast_convert_type(probe_pat, jnp.float32)

    def count_pass(src, probe_f, with_sum):
        def body(i, carry):
            if with_sum:
                cacc, acc = carry
            else:
                cacc = carry
            b = i * (L * UC)
            for u in range(UC):
                v = src[pl.ds(b + u * L, L)]
                cacc = cacc + plsc.all_reduce_population_count(v <= probe_f)
                if with_sum:
                    acc = acc + v
            return (cacc, acc) if with_sum else cacc

        init = (zero_i, jnp.zeros((L,), jnp.float32)) if with_sum else zero_i
        out = lax.fori_loop(0, SEG // UC, body, init)
        if with_sum:
            return out[0][0], out[1]
        return out[0], None

    def extract(src, dst, lo_f, hi_f):
        def body(i, off):
            b = i * (L * U)
            for u in range(U):
                v = src[pl.ds(b + u * L, L)]
                m = jnp.logical_and(v >= lo_f, v <= hi_f)
                plsc.store_scatter(dst, [off], v, mask=m)
                off = off + jnp.where(m, one_i, zero_i)
            return off

        off = lax.fori_loop(0, SEG // U, body, base_v)
        return off - base_v

    def splitn(src, d_lo, d_hi, s_vec, probe_f):
        trip = plsc.cummax(s_vec)[L - 1]

        def body(i, carry):
            offL, offH = carry
            for u in range(U):
                iu = i * U + u
                idx = base_v + iu
                v = plsc.load_gather(src, [idx])
                valid = s_vec > iu
                le0 = v <= probe_f
                m = jnp.logical_and(le0, valid)
                mh = jnp.logical_and(jnp.logical_not(le0), valid)
                plsc.store_scatter(d_lo, [offL], v, mask=m)
                plsc.store_scatter(d_hi, [offH], v, mask=mh)
                offL = offL + jnp.where(m, one_i, zero_i)
                offH = offH + jnp.where(mh, one_i, zero_i)
            return (offL, offH)

        nit = (trip + (U - 1)) // U
        offL, offH = lax.fori_loop(0, nit, body, (base_v, base_v))
        return offL - base_v, offH - base_v

    def split_from(src_id, s_vec, probe_f):
        return lax.cond(
            src_id == 0,
            lambda: splitn(buf0, buf1, buf2, s_vec, probe_f),
            lambda: lax.cond(
                src_id == 1,
                lambda: splitn(buf1, buf0, buf2, s_vec, probe_f),
                lambda: splitn(buf2, buf0, buf1, s_vec, probe_f)))

    def collect(src_id, s_vec, size_s, k_s):
        def coll(src):
            off = jnp.int32(0)
            for i in range(L):
                v = plsc.load_gather(src, [base_v + i])
                valid = s_vec > i
                plsc.store_compressed(tiny_v.at[pl.ds(off, L)], v,
                                      mask=valid)
                off = off + plsc.all_reduce_population_count(valid)[0]
            w = tiny_v[pl.ds(0, L)]
            valid2 = lane < jnp.broadcast_to(size_s, (L,))
            skeys, _, _ = plsc.sort_key_val(w, w, mask=valid2)
            sel = jnp.where(lane == jnp.broadcast_to(k_s, (L,)), skeys, 0.0)
            return _xsum(sel)
        return lax.cond(
            src_id == 0, lambda: coll(buf0),
            lambda: lax.cond(src_id == 1, lambda: coll(buf1),
                             lambda: coll(buf2)))

    def process_row(jj, res):
        row = wid * RPW + jj
        pltpu.sync_copy(x_hbm.at[row], buf0.at[pl.ds(0, N)])

        lo_v = zero_i
        hi_v = jnp.full((L,), HI0, jnp.int32)
        kk = jnp.int32(K)
        below = jnp.int32(0)
        mean = jnp.float32(0.0)
        for p in range(NCOUNT):
            if p == 0:
                probe_pat = jnp.full((L,), 0x3F000000, jnp.int32)
                probe_f = jnp.full((L,), 0.5, jnp.float32)
                c_le, acc = count_pass(buf0, probe_f, True)
                mean = _xsum(acc) * (1.0 / N)
            else:
                probe_pat, probe_f = mk_probe(lo_v, hi_v)
                c_le, _ = count_pass(buf0, probe_f, False)
            cin = c_le - below
            go = kk < cin
            go_v = jnp.broadcast_to(go, (L,))
            hi_v = jnp.where(go_v, probe_pat, hi_v)
            lo_v = jnp.where(go_v, lo_v, probe_pat + 1)
            kk = jnp.where(go, kk, kk - cin)
            below = jnp.where(go, below, c_le)

        s1 = extract(buf0, buf1,
                     lax.bitcast_convert_type(lo_v, jnp.float32),
                     lax.bitcast_convert_type(hi_v, jnp.float32))
        size1 = _xsum(s1)

        def w_cond(st):
            src_id, s_vec, size_s, k_s, it, lo_v, hi_v = st
            return (size_s > L) & (lo_v[0] < hi_v[0]) & (it < MAXIT)

        def w_body(st):
            src_id, s_vec, size_s, k_s, it, lo_v, hi_v = st
            probe_pat, probe_f = mk_probe(lo_v, hi_v)
            sL2, sH2 = split_from(src_id, s_vec, probe_f)
            cL2 = _xsum(sL2)
            go2 = k_s < cL2
            go_v = jnp.broadcast_to(go2, (L,))
            k2 = jnp.where(go2, k_s, k_s - cL2)
            size2 = jnp.where(go2, cL2, size_s - cL2)
            s2 = jnp.where(go_v, sL2, sH2)
            hi2 = jnp.where(go_v, probe_pat, hi_v)
            lo2 = jnp.where(go_v, lo_v, probe_pat + 1)
            lodest = jnp.where(src_id == 0, 1, 0)
            hidest = jnp.where(src_id == 2, 1, 2)
            src2 = jnp.where(go2, lodest, hidest)
            return (src2, s2, size2, k2, it + 1, lo2, hi2)

        src_id, s_vec, size_s, k_s, _, lo_v, _ = lax.while_loop(
            w_cond, w_body,
            (jnp.int32(1), s1, size1, kk, jnp.int32(0), lo_v, hi_v))

        m_sorted = lax.cond(
            size_s <= L,
            lambda: collect(src_id, s_vec, size_s, k_s),
            lambda: jnp.float32(0.0))
        m_bit = jnp.where(size_s <= L, m_sorted,
                          lax.bitcast_convert_type(lo_v, jnp.float32)[0])

        margin = jnp.abs(m_bit - 0.5)
        md = mean * margin
        rep = jnp.where(m_bit > 0.5, 0.5 + md, m_bit + md)
        return jnp.where(lane == jnp.broadcast_to(jj, (L,)),
                         jnp.broadcast_to(rep, (L,)), res)

    res = lax.fori_loop(0, RPW, process_row, jnp.zeros((L,), jnp.float32))
    res_v[...] = res
    pltpu.sync_copy(res_v, out_hbm.at[wid])


def kernel(x):
    padded = _soft_majority_sc(x)
    return padded[:, :RPW].reshape(R)
